# Initial kernel scaffold; baseline (speedup 1.0000x reference)
import jax
import jax.numpy as jnp
from jax.experimental import pallas as pl


def kernel(in_node_feats, pW1, pb1, pW2, pb2, eW1, eb1, eW2, eb2, conv_b, gW_ih, gW_hh, gb_ih, gb_hh):
    raise NotImplementedError("write your pallas kernel here")


# speedup vs baseline: 17.5380x; 17.5380x over previous
import numpy as np
import jax
import jax.numpy as jnp
from jax.experimental import pallas as pl

_B = 2
_NT = 6
_NX = 64
_DIN = 128
_HID = 16
_N = _NX * _NX
_STEPS = 3

_REL4 = np.array([[-1.0, 0.0], [1.0, 0.0], [0.0, -1.0], [0.0, 1.0]], np.float32)


def _body(x_ref, pW1t_ref, pb1_ref, pW2t_ref, pb2_ref, th_ref, cb_ref,
          giW_ref, ghW_ref, gib_ref, ghb_ref, out_ref):
    x = x_ref[0]
    nf = jnp.maximum(
        jnp.dot(x, pW1t_ref[...], preferred_element_type=jnp.float32)
        + pb1_ref[...], 0.0)
    h = jnp.dot(nf, pW2t_ref[...], preferred_element_type=jnp.float32) + pb2_ref[...]

    n_idx = jax.lax.broadcasted_iota(jnp.int32, (_N, 1), 0)
    jj = n_idx % _NX
    ii = n_idx // _NX
    m_right = (jj < _NX - 1).astype(jnp.float32)
    m_left = (jj > 0).astype(jnp.float32)
    m_down = (ii < _NX - 1).astype(jnp.float32)
    m_up = (ii > 0).astype(jnp.float32)
    invdeg = 1.0 / (m_right + m_left + m_down + m_up)

    zrow64 = jnp.zeros((_NX, _HID), jnp.float32)
    zrow1 = jnp.zeros((1, _HID), jnp.float32)
    cb = cb_ref[...]
    gib = gib_ref[...]
    ghb = ghb_ref[...]

    for _ in range(_STEPS):
        y0 = jnp.dot(h, th_ref[0], preferred_element_type=jnp.float32)
        y1 = jnp.dot(h, th_ref[1], preferred_element_type=jnp.float32)
        y2 = jnp.dot(h, th_ref[2], preferred_element_type=jnp.float32)
        y3 = jnp.dot(h, th_ref[3], preferred_element_type=jnp.float32)
        c0 = jnp.concatenate([y0[_NX:], zrow64], axis=0)
        c1 = jnp.concatenate([zrow64, y1[:-_NX]], axis=0)
        c2 = jnp.concatenate([y2[1:], zrow1], axis=0) * m_right
        c3 = jnp.concatenate([zrow1, y3[:-1]], axis=0) * m_left
        agg = (c0 + c1 + c2 + c3) * invdeg + cb
        conv = jnp.maximum(agg, 0.0)

        gi = jnp.dot(conv, giW_ref[...], preferred_element_type=jnp.float32) + gib
        gh = jnp.dot(h, ghW_ref[...], preferred_element_type=jnp.float32) + ghb
        r = jax.nn.sigmoid(gi[:, :_HID] + gh[:, :_HID])
        z = jax.nn.sigmoid(gi[:, _HID:2 * _HID] + gh[:, _HID:2 * _HID])
        nn = jnp.tanh(gi[:, 2 * _HID:] + r * gh[:, 2 * _HID:])
        h = (1.0 - z) * nn + z * h

    out_ref[0] = h


def kernel(in_node_feats, pW1, pb1, pW2, pb2, eW1, eb1, eW2, eb2, conv_b,
           gW_ih, gW_hh, gb_ih, gb_hh):
    eh = jax.nn.relu(jnp.asarray(_REL4) @ eW1.T + eb1)
    theta = (eh @ eW2.T + eb2).reshape(4, _HID, _HID)

    x = in_node_feats.reshape(_B * _NT, _N, _DIN)
    full = lambda shape: pl.BlockSpec(shape, lambda t: (0,) * len(shape))
    out = pl.pallas_call(
        _body,
        grid=(_B * _NT,),
        in_specs=[
            pl.BlockSpec((1, _N, _DIN), lambda t: (t, 0, 0)),
            full((_DIN, _HID)),
            full((1, _HID)),
            full((_HID, _HID)),
            full((1, _HID)),
            full((4, _HID, _HID)),
            full((1, _HID)),
            full((_HID, 3 * _HID)),
            full((_HID, 3 * _HID)),
            full((1, 3 * _HID)),
            full((1, 3 * _HID)),
        ],
        out_specs=pl.BlockSpec((1, _N, _HID), lambda t: (t, 0, 0)),
        out_shape=jax.ShapeDtypeStruct((_B * _NT, _N, _HID), jnp.float32),
    )(x, pW1.T, pb1.reshape(1, -1), pW2.T, pb2.reshape(1, -1), theta,
      conv_b.reshape(1, -1), gW_ih.T, gW_hh.T, gb_ih.reshape(1, -1),
      gb_hh.reshape(1, -1))
    return out.reshape(_B, _NT, _NX, _NX, _HID)
